# baseline (speedup 1.0000x reference)
import functools

import jax
import jax.numpy as jnp
from jax import lax
from jax.experimental import pallas as pl
from jax.experimental.pallas import tpu as pltpu
from jax.experimental.pallas import tpu_sc as plsc

B, S, P, L = 16, 256, 8192, 16
D_TOK, D_OH, D_DEP = 256, 50, 50
D = D_TOK + D_OH + D_DEP
DP = 384
DPW = 256
H = 128
PB = 512
C = 2
PC = P // C

NC, NS = 2, 16
NW = NC * NS
KCH = 128
NB = 2


def _gather_body(rw, idx_hbm, table_hbm, out_hbm, idx_v, rows_v, semg, sems):
    wid = lax.axis_index("s") * NC + lax.axis_index("c")
    base = wid * rw
    pltpu.sync_copy(idx_hbm.at[pl.ds(base, rw)], idx_v)

    def rnd(r, carry):
        for b in range(NB):
            @pl.when(r > 0)
            def _wait_prev_scatter():
                pltpu.make_async_copy(
                    rows_v.at[b], out_hbm.at[pl.ds(base, KCH)], sems).wait()
            c = r * NB + b
            pltpu.async_copy(
                table_hbm.at[idx_v.at[pl.ds(c * KCH, KCH)]], rows_v.at[b],
                semg)
        for b in range(NB):
            c = r * NB + b
            pltpu.make_async_copy(
                table_hbm.at[idx_v.at[pl.ds(c * KCH, KCH)]], rows_v.at[b],
                semg).wait()
            pltpu.async_copy(
                rows_v.at[b], out_hbm.at[pl.ds(base + c * KCH, KCH)], sems)
        return carry

    lax.fori_loop(0, rw // (KCH * NB), rnd, 0)
    for b in range(NB):
        pltpu.make_async_copy(
            rows_v.at[b], out_hbm.at[pl.ds(base, KCH)], sems).wait()


def _sc_gather(flat_idx, table):
    n_rows = flat_idx.shape[0]
    rw = n_rows // NW
    mesh = plsc.VectorSubcoreMesh(core_axis_name="c", subcore_axis_name="s")
    return pl.kernel(
        functools.partial(_gather_body, rw),
        out_type=jax.ShapeDtypeStruct((n_rows, DPW), jnp.float32),
        mesh=mesh,
        scratch_types=[
            pltpu.VMEM((rw,), jnp.int32),
            pltpu.VMEM((NB, KCH, DPW), jnp.float32),
            pltpu.SemaphoreType.DMA,
            pltpu.SemaphoreType.DMA,
        ],
    )(flat_idx, table)


def _scan_body(x_ref, wa0_ref, wb0_ref, u0_ref, wa1_ref, wb1_ref, u1_ref,
               out_ref):
    f32 = jnp.float32
    bf16 = jnp.bfloat16
    HN = 2
    HPB = PB // HN

    def cell(x_t, h, c, wa_ref, wb_ref, u_ref):
        xi = lax.bitcast_convert_type(x_t, jnp.int32)
        xa = lax.bitcast_convert_type(xi << 16, f32).astype(bf16)
        xo = lax.bitcast_convert_type(xi[:, :H] & jnp.int32(-65536),
                                      f32).astype(bf16)
        z = jnp.dot(xa, wa_ref[...], preferred_element_type=f32)
        z += jnp.dot(xo, wb_ref[...], preferred_element_type=f32)
        z += jnp.dot(h.astype(bf16), u_ref[...], preferred_element_type=f32)
        i = 0.5 * jnp.tanh(z[:, 0 * H:1 * H]) + 0.5
        o = 0.5 * jnp.tanh(z[:, 1 * H:2 * H]) + 0.5
        u = jnp.tanh(z[:, 2 * H:3 * H])
        f = 0.5 * jnp.tanh(z[:, 3 * H:4 * H]) + 0.5
        c_new = i * u + f * c
        h_new = o * jnp.tanh(c_new)
        return h_new, c_new

    zero = jnp.zeros((HPB, H), f32)
    st = [[zero, zero, zero, zero, zero] for _ in range(HN)]
    for t in range(L):
        for g in range(HN):
            r = slice(g * HPB, (g + 1) * HPB)
            s = st[g]
            s[0], s[1] = cell(x_ref[t, r], s[0], s[1],
                              wa0_ref, wb0_ref, u0_ref)
            s[2], s[3] = cell(x_ref[L - 1 - t, r], s[2], s[3],
                              wa1_ref, wb1_ref, u1_ref)
            if t == 0:
                s[4] = s[2]

    for g in range(HN):
        r = slice(g * HPB, (g + 1) * HPB)
        out_ref[r, 0 * H:1 * H] = st[g][0]
        out_ref[r, 1 * H:2 * H] = st[g][2]
        out_ref[r, 2 * H:3 * H] = st[g][4]


def _lstm_scan(x, wa0, wb0, u0, wa1, wb1, u1, interpret=False):
    grid = (PC // PB,)
    wspec = lambda n: pl.BlockSpec((n, 4 * H), lambda i: (0, 0))
    return pl.pallas_call(
        _scan_body,
        grid=grid,
        in_specs=[
            pl.BlockSpec((L, PB, DPW), lambda i: (0, i, 0)),
            wspec(2 * H), wspec(H), wspec(H),
            wspec(2 * H), wspec(H), wspec(H),
        ],
        out_specs=pl.BlockSpec((PB, 3 * H), lambda i: (i, 0)),
        out_shape=jax.ShapeDtypeStruct((PC, 3 * H), jnp.float32),
        interpret=interpret,
    )(x, wa0, wb0, u0, wa1, wb1, u1)


def kernel(token_embs, dep_embs, one_hot_embs, roots, token_mask, deplinks,
           path_idx, path_batch,
           W_iou0, U_iou0, b_iou0, W_f0, U_f0, b_f0,
           W_iou1, U_iou1, b_iou1, W_f1, U_f1, b_f1):
    f32 = jnp.float32
    bf16 = jnp.bfloat16
    node = jnp.concatenate((token_embs, one_hot_embs, dep_embs), axis=-1)
    node = node * token_mask[..., None].astype(f32)
    table = jnp.pad(node.reshape(B * S, D), ((0, 0), (0, 2 * DPW - D)))
    table = table.at[:, D].set(1.0).astype(bf16)
    table_w = lax.bitcast_convert_type(
        jnp.stack((table[:, :DPW], table[:, DPW:]), axis=-1), f32)

    gate_scale = jnp.concatenate((jnp.full((2 * H,), 0.5), jnp.ones((H,)),
                                  jnp.full((H,), 0.5)))[None, :]

    def wcat(Wiou, Wf, biou, bf):
        w = jnp.pad(jnp.concatenate((Wiou, Wf), axis=1),
                    ((0, DP - D), (0, 0)))
        w = w.at[D, :].set(jnp.concatenate((biou, bf)))
        w = (w * gate_scale).astype(bf16)
        return w[:2 * H], w[2 * H:]

    wa0, wb0 = wcat(W_iou0, W_f0, b_iou0, b_f0)
    wa1, wb1 = wcat(W_iou1, W_f1, b_iou1, b_f1)
    u0 = (jnp.concatenate((U_iou0, U_f0), axis=1) * gate_scale).astype(bf16)
    u1 = (jnp.concatenate((U_iou1, U_f1), axis=1) * gate_scale).astype(bf16)

    flat_idx = (path_batch[None, :] * S + path_idx.T).astype(jnp.int32)
    outs = []
    for c in range(C):
        idx_c = flat_idx[:, c * PC:(c + 1) * PC].reshape(-1)
        x_c = _sc_gather(idx_c, table_w).reshape(L, PC, DPW)
        outs.append(_lstm_scan(x_c, wa0, wb0, u0, wa1, wb1, u1))
    return jnp.concatenate(outs, axis=0)


# speedup vs baseline: 1.0587x; 1.0587x over previous
import functools

import jax
import jax.numpy as jnp
from jax import lax
from jax.experimental import pallas as pl
from jax.experimental.pallas import tpu as pltpu
from jax.experimental.pallas import tpu_sc as plsc

B, S, P, L = 16, 256, 8192, 16
D_TOK, D_OH, D_DEP = 256, 50, 50
D = D_TOK + D_OH + D_DEP
DP = 384
DPW = 256
H = 128
PB = 512
C = 4
PC = P // C

NC, NS = 2, 16
NW = NC * NS
KCH = 128
NB = 2


def _gather_body(rw, idx_hbm, table_hbm, out_hbm, idx_v, rows_v, semg, sems):
    wid = lax.axis_index("s") * NC + lax.axis_index("c")
    base = wid * rw
    pltpu.sync_copy(idx_hbm.at[pl.ds(base, rw)], idx_v)

    def rnd(r, carry):
        for b in range(NB):
            @pl.when(r > 0)
            def _wait_prev_scatter():
                pltpu.make_async_copy(
                    rows_v.at[b], out_hbm.at[pl.ds(base, KCH)], sems).wait()
            c = r * NB + b
            pltpu.async_copy(
                table_hbm.at[idx_v.at[pl.ds(c * KCH, KCH)]], rows_v.at[b],
                semg)
        for b in range(NB):
            c = r * NB + b
            pltpu.make_async_copy(
                table_hbm.at[idx_v.at[pl.ds(c * KCH, KCH)]], rows_v.at[b],
                semg).wait()
            pltpu.async_copy(
                rows_v.at[b], out_hbm.at[pl.ds(base + c * KCH, KCH)], sems)
        return carry

    lax.fori_loop(0, rw // (KCH * NB), rnd, 0)
    for b in range(NB):
        pltpu.make_async_copy(
            rows_v.at[b], out_hbm.at[pl.ds(base, KCH)], sems).wait()


def _sc_gather(flat_idx, table):
    n_rows = flat_idx.shape[0]
    rw = n_rows // NW
    mesh = plsc.VectorSubcoreMesh(core_axis_name="c", subcore_axis_name="s")
    return pl.kernel(
        functools.partial(_gather_body, rw),
        out_type=jax.ShapeDtypeStruct((n_rows, DPW), jnp.float32),
        mesh=mesh,
        scratch_types=[
            pltpu.VMEM((rw,), jnp.int32),
            pltpu.VMEM((NB, KCH, DPW), jnp.float32),
            pltpu.SemaphoreType.DMA,
            pltpu.SemaphoreType.DMA,
        ],
    )(flat_idx, table)


def _pack_body(lo_ref, hi_ref, out_ref):
    i32 = jnp.int32
    f32 = jnp.float32
    bf16 = jnp.bfloat16
    lo = lax.bitcast_convert_type(lo_ref[...].astype(bf16).astype(f32), i32)
    hi = lax.bitcast_convert_type(hi_ref[...].astype(bf16).astype(f32), i32)
    word = lax.shift_right_logical(lo, 16) | (hi & jnp.int32(-65536))
    out_ref[...] = lax.bitcast_convert_type(word, f32)


def _pack_table(lo, hi):
    return pl.pallas_call(
        _pack_body,
        out_shape=jax.ShapeDtypeStruct((B * S, DPW), jnp.float32),
    )(lo, hi)


def _scan_body(x_ref, prev_ref, wa0_ref, wb0_ref, u0_ref, wa1_ref, wb1_ref,
               u1_ref, out_ref):
    f32 = jnp.float32
    bf16 = jnp.bfloat16
    HN = 2
    HPB = PB // HN

    def cell(x_t, h, c, wa_ref, wb_ref, u_ref):
        xi = lax.bitcast_convert_type(x_t, jnp.int32)
        xa = lax.bitcast_convert_type(xi << 16, f32).astype(bf16)
        xo = lax.bitcast_convert_type(xi[:, :H] & jnp.int32(-65536),
                                      f32).astype(bf16)
        z = jnp.dot(xa, wa_ref[...], preferred_element_type=f32)
        z += jnp.dot(xo, wb_ref[...], preferred_element_type=f32)
        z += jnp.dot(h.astype(bf16), u_ref[...], preferred_element_type=f32)
        i = 0.5 * jnp.tanh(z[:, 0 * H:1 * H]) + 0.5
        o = 0.5 * jnp.tanh(z[:, 1 * H:2 * H]) + 0.5
        u = jnp.tanh(z[:, 2 * H:3 * H])
        f = 0.5 * jnp.tanh(z[:, 3 * H:4 * H]) + 0.5
        c_new = i * u + f * c
        h_new = o * jnp.tanh(c_new)
        return h_new, c_new

    zero = jnp.zeros((HPB, H), f32)
    st = [[zero, zero, zero, zero, zero] for _ in range(HN)]
    for t in range(L):
        for g in range(HN):
            r = slice(g * HPB, (g + 1) * HPB)
            s = st[g]
            s[0], s[1] = cell(x_ref[t, r], s[0], s[1],
                              wa0_ref, wb0_ref, u0_ref)
            s[2], s[3] = cell(x_ref[L - 1 - t, r], s[2], s[3],
                              wa1_ref, wb1_ref, u1_ref)
            if t == 0:
                s[4] = s[2]

    for g in range(HN):
        r = slice(g * HPB, (g + 1) * HPB)
        out_ref[r, 0 * H:1 * H] = st[g][0]
        out_ref[r, 1 * H:2 * H] = st[g][2]
        out_ref[r, 2 * H:3 * H] = st[g][4]


def _lstm_scan(x, prev, wa0, wb0, u0, wa1, wb1, u1, c, interpret=False):
    grid = (PC // PB,)
    base = c * (PC // PB)
    wspec = lambda n: pl.BlockSpec((n, 4 * H), lambda i: (0, 0))
    have_prev = prev is not None
    prev_args = (prev,) if have_prev else ()
    prev_spec = [pl.BlockSpec(memory_space=pl.ANY)] if have_prev else []
    return pl.pallas_call(
        (lambda xr, pr, *a: _scan_body(xr, pr, *a)) if have_prev else
        (lambda xr, *a: _scan_body(xr, None, *a)),
        grid=grid,
        in_specs=[
            pl.BlockSpec((L, PB, DPW), lambda i: (0, i, 0)),
            *prev_spec,
            wspec(2 * H), wspec(H), wspec(H),
            wspec(2 * H), wspec(H), wspec(H),
        ],
        out_specs=pl.BlockSpec((PB, 3 * H), lambda i: (base + i, 0)),
        out_shape=jax.ShapeDtypeStruct((P, 3 * H), jnp.float32),
        input_output_aliases={1: 0} if have_prev else {},
        interpret=interpret,
    )(x, *prev_args, wa0, wb0, u0, wa1, wb1, u1)


def kernel(token_embs, dep_embs, one_hot_embs, roots, token_mask, deplinks,
           path_idx, path_batch,
           W_iou0, U_iou0, b_iou0, W_f0, U_f0, b_f0,
           W_iou1, U_iou1, b_iou1, W_f1, U_f1, b_f1):
    f32 = jnp.float32
    bf16 = jnp.bfloat16
    mcol = token_mask.reshape(B * S, 1).astype(f32)
    lo = token_embs.reshape(B * S, DPW) * mcol
    hi = jnp.concatenate(
        (jnp.concatenate((one_hot_embs, dep_embs), axis=-1)
         .reshape(B * S, D - DPW) * mcol,
         jnp.ones((B * S, 1), f32),
         jnp.zeros((B * S, 2 * DPW - D - 1), f32)), axis=-1)
    table_w = _pack_table(lo, hi)

    gate_scale = jnp.concatenate((jnp.full((2 * H,), 0.5), jnp.ones((H,)),
                                  jnp.full((H,), 0.5)))[None, :]

    def wcat(Wiou, Wf, biou, bf):
        w = jnp.pad(jnp.concatenate((Wiou, Wf), axis=1),
                    ((0, DP - D), (0, 0)))
        w = w.at[D, :].set(jnp.concatenate((biou, bf)))
        w = (w * gate_scale).astype(bf16)
        return w[:2 * H], w[2 * H:]

    wa0, wb0 = wcat(W_iou0, W_f0, b_iou0, b_f0)
    wa1, wb1 = wcat(W_iou1, W_f1, b_iou1, b_f1)
    u0 = (jnp.concatenate((U_iou0, U_f0), axis=1) * gate_scale).astype(bf16)
    u1 = (jnp.concatenate((U_iou1, U_f1), axis=1) * gate_scale).astype(bf16)

    flat_idx = (path_batch[None, :] * S + path_idx.T).astype(jnp.int32)
    out = None
    for c in range(C):
        idx_c = flat_idx[:, c * PC:(c + 1) * PC].reshape(-1)
        x_c = _sc_gather(idx_c, table_w).reshape(L, PC, DPW)
        out = _lstm_scan(x_c, out, wa0, wb0, u0, wa1, wb1, u1, c)
    return out
